# Initial kernel scaffold; baseline (speedup 1.0000x reference)
import jax
import jax.numpy as jnp
from jax.experimental import pallas as pl


def kernel(x, emb_weight):
    raise NotImplementedError("write your pallas kernel here")


# speedup vs baseline: 2.5313x; 2.5313x over previous
import functools

import jax
import jax.numpy as jnp
from jax import lax
from jax.experimental import pallas as pl
from jax.experimental.pallas import tpu as pltpu
from jax.experimental.pallas import tpu_sc as plsc

_BATCH, _HIST, _D = 16384, 200, 16
_TOTAL = _BATCH * _HIST
_NC, _NS = 2, 16
_NW = _NC * _NS
_PER_W = _TOTAL // _NW
_CHUNK = 2048
_NBUF = 2
_NCHUNK = _PER_W // _CHUNK
_NOUTER = _NCHUNK // _NBUF


def _make_gather():
    mesh = plsc.VectorSubcoreMesh(core_axis_name="c", subcore_axis_name="s")

    @functools.partial(
        pl.kernel,
        mesh=mesh,
        compiler_params=pltpu.CompilerParams(use_tc_tiling_on_sc=False),
        out_type=jax.ShapeDtypeStruct((_TOTAL, _D), jnp.float32),
        scratch_types=[
            pltpu.VMEM((_NBUF, _CHUNK), jnp.int32),
            pltpu.VMEM((_NBUF, _CHUNK, _D), jnp.float32),
            pltpu.SemaphoreType.DMA,
            pltpu.SemaphoreType.DMA,
            pltpu.SemaphoreType.DMA,
            pltpu.SemaphoreType.DMA,
            pltpu.SemaphoreType.DMA,
            pltpu.SemaphoreType.DMA,
        ],
    )
    def k(table, idx, out, idx_v, rows_v, i0, i1, g0, g1, o0, o1):
        isem = (i0, i1)
        gsem = (g0, g1)
        osem = (o0, o1)
        wid = lax.axis_index("s") * _NC + lax.axis_index("c")
        base = wid * _PER_W

        def idx_copy(c, b):
            return pltpu.make_async_copy(
                idx.at[pl.ds(base + c * _CHUNK, _CHUNK)], idx_v.at[b], isem[b])

        def gat_copy(b):
            return pltpu.make_async_copy(
                table.at[idx_v.at[b]], rows_v.at[b], gsem[b])

        def out_copy(c, b):
            return pltpu.make_async_copy(
                rows_v.at[b], out.at[pl.ds(base + c * _CHUNK, _CHUNK)], osem[b])

        for b in range(_NBUF):
            idx_copy(b, b).start()

        def body(go, carry):
            for b in range(_NBUF):
                c = go * _NBUF + b
                idx_copy(c, b).wait()

                @pl.when(go > 0)
                def _():
                    out_copy(c, b).wait()

                gat_copy(b).start()
                gat_copy(b).wait()

                @pl.when(go < _NOUTER - 1)
                def _():
                    idx_copy(c + _NBUF, b).start()

                out_copy(c, b).start()
            return carry

        lax.fori_loop(0, _NOUTER, body, 0)
        for b in range(_NBUF):
            out_copy(_NCHUNK - _NBUF + b, b).wait()

    return k


_gather = _make_gather()


def kernel(x, emb_weight):
    idx = x.reshape(-1).astype(jnp.int32)
    out = _gather(emb_weight, idx)
    return out.reshape(_BATCH, _HIST, _D)
